# baseline (speedup 1.0000x reference)
import functools

import jax
import jax.numpy as jnp
from jax import lax
from jax.experimental import pallas as pl
from jax.experimental.pallas import tpu as pltpu
from jax.experimental.pallas import tpu_sc as plsc

NW = 32
CHUNK = 128
LANES = 16


def _fractal(c):
    z = c
    for _ in range(9):
        z = z * z + c
    return z


def _build(total, vocab, dim, n_chunks):
    per_w = n_chunks * CHUNK
    mesh = plsc.VectorSubcoreMesh(core_axis_name="c", subcore_axis_name="s")

    @functools.partial(
        pl.kernel,
        mesh=mesh,
        compiler_params=pltpu.CompilerParams(use_tc_tiling_on_sc=False),
        out_type=jax.ShapeDtypeStruct((total, dim), jnp.float32),
        scratch_types=[
            pltpu.VMEM((n_chunks, CHUNK), jnp.int32),
            pltpu.VMEM((CHUNK, dim), jnp.float32),
            pltpu.SemaphoreType.DMA,
        ],
    )
    def fractal_gather(idx_hbm, table_hbm, out_hbm, idx_v, rows_v, sem):
        wid = lax.axis_index("s") * 2 + lax.axis_index("c")
        pltpu.sync_copy(idx_hbm.at[wid], idx_v)

        def chunk_body(j, carry):
            pltpu.async_copy(table_hbm.at[idx_v.at[j]], rows_v, sem).wait()

            def row_body(r, c2):
                for s in range(0, dim, LANES):
                    c = rows_v[r, pl.ds(s, LANES)]
                    rows_v[r, pl.ds(s, LANES)] = _fractal(c)
                return c2

            lax.fori_loop(0, CHUNK, row_body, 0)
            base = wid * per_w + j * CHUNK
            pltpu.sync_copy(rows_v, out_hbm.at[pl.ds(base, CHUNK)])
            return carry

        lax.fori_loop(0, n_chunks, chunk_body, 0)

    return fractal_gather


def kernel(token_id, weights):
    batch, hist = token_id.shape
    vocab, dim = weights.shape
    total = batch * hist
    assert total % (NW * CHUNK) == 0 and dim % LANES == 0
    n_chunks = total // (NW * CHUNK)
    idx = token_id.reshape(NW, n_chunks, CHUNK).astype(jnp.int32)
    out = _build(total, vocab, dim, n_chunks)(idx, weights)
    return out.reshape(batch, hist, dim)


# speedup vs baseline: 1.1694x; 1.1694x over previous
import functools

import jax
import jax.numpy as jnp
from jax import lax
from jax.experimental import pallas as pl
from jax.experimental.pallas import tpu as pltpu
from jax.experimental.pallas import tpu_sc as plsc

NW = 32
CHUNK = 128
LANES = 16
NBUF = 5
ROWS_PER_ITER = 4


def _fractal(c):
    z = c
    for _ in range(9):
        z = z * z + c
    return z


def _build(total, vocab, dim, n_chunks):
    per_w = n_chunks * CHUNK
    n_outer = n_chunks // NBUF
    mesh = plsc.VectorSubcoreMesh(core_axis_name="c", subcore_axis_name="s")

    @functools.partial(
        pl.kernel,
        mesh=mesh,
        compiler_params=pltpu.CompilerParams(use_tc_tiling_on_sc=False),
        out_type=jax.ShapeDtypeStruct((total, dim), jnp.float32),
        scratch_types=[
            pltpu.VMEM((n_chunks, CHUNK), jnp.int32),
            pltpu.VMEM((NBUF, CHUNK, dim), jnp.float32),
            pltpu.VMEM((NBUF, CHUNK, dim), jnp.float32),
        ]
        + [pltpu.SemaphoreType.DMA] * (2 * NBUF),
    )
    def fractal_gather(idx_hbm, table_hbm, out_hbm, idx_v, in_v, out_v, *sems):
        gsems, ssems = sems[:NBUF], sems[NBUF:]
        wid = lax.axis_index("s") * 2 + lax.axis_index("c")
        pltpu.sync_copy(idx_hbm.at[wid], idx_v)
        for b in range(NBUF):
            pltpu.async_copy(table_hbm.at[idx_v.at[b]], in_v.at[b], gsems[b])

        def outer(g, carry):
            for b in range(NBUF):
                j = g * NBUF + b
                pltpu.make_async_copy(
                    table_hbm.at[idx_v.at[j]], in_v.at[b], gsems[b]
                ).wait()

                @pl.when(g > 0)
                def _wait_store():
                    pltpu.make_async_copy(
                        out_v.at[b], out_hbm.at[pl.ds(0, CHUNK)], ssems[b]
                    ).wait()

                @plsc.parallel_loop(0, CHUNK, step=ROWS_PER_ITER)
                def _compute(r):
                    for rr in range(ROWS_PER_ITER):
                        for s in range(0, dim, LANES):
                            c = in_v[b, r + rr, pl.ds(s, LANES)]
                            out_v[b, r + rr, pl.ds(s, LANES)] = _fractal(c)

                @pl.when(j + NBUF < n_chunks)
                def _next_gather():
                    pltpu.async_copy(
                        table_hbm.at[idx_v.at[j + NBUF]], in_v.at[b], gsems[b]
                    )

                base = wid * per_w + j * CHUNK
                pltpu.async_copy(
                    out_v.at[b], out_hbm.at[pl.ds(base, CHUNK)], ssems[b]
                )
            return carry

        lax.fori_loop(0, n_outer, outer, 0)
        for b in range(NBUF):
            pltpu.make_async_copy(
                out_v.at[b], out_hbm.at[pl.ds(0, CHUNK)], ssems[b]
            ).wait()

    return fractal_gather


def kernel(token_id, weights):
    batch, hist = token_id.shape
    vocab, dim = weights.shape
    total = batch * hist
    assert total % (NW * CHUNK) == 0 and dim % LANES == 0
    n_chunks = total // (NW * CHUNK)
    assert n_chunks % NBUF == 0
    idx = token_id.reshape(NW, n_chunks, CHUNK).astype(jnp.int32)
    out = _build(total, vocab, dim, n_chunks)(idx, weights)
    return out.reshape(batch, hist, dim)
